# baseline (speedup 1.0000x reference)
import functools

import jax
import jax.numpy as jnp
from jax import lax
from jax.experimental import pallas as pl
from jax.experimental.pallas import tpu as pltpu


def _gram_kernel(feat_ref, out_ref, buf_ref, *, nsteps, scale):
    kk = pl.program_id(0)

    @pl.when(kk == 0)
    def _():
        out_ref[...] = jnp.zeros_like(out_ref)
        buf_ref[1] = jnp.zeros_like(buf_ref[1])

    f = feat_ref[...].astype(jnp.bfloat16)
    buf_ref[kk % 2] = f.reshape(f.shape[0], f.shape[1] * f.shape[2])

    g = buf_ref[(kk + 1) % 2]
    out_ref[...] += lax.dot_general(
        g, g,
        dimension_numbers=(((1,), (1,)), ((), ())),
        preferred_element_type=jnp.float32,
    )

    @pl.when(kk == nsteps)
    def _():
        out_ref[...] = out_ref[...] * scale


def kernel(x):
    b, c, h, w = x.shape
    m = b * c
    feats = x.reshape(m, h, w)
    scale = 1.0 / float(b * c * h * w)

    th = 16
    while th > 1 and h % th:
        th //= 2
    steps = h // th

    return pl.pallas_call(
        functools.partial(_gram_kernel, nsteps=steps, scale=scale),
        out_shape=jax.ShapeDtypeStruct((m, m), jnp.float32),
        grid=(steps + 1,),
        in_specs=[
            pl.BlockSpec((m, th, w),
                         lambda kk, ns=steps: (0, jnp.minimum(kk, ns - 1), 0))
        ],
        out_specs=pl.BlockSpec((m, m), lambda kk: (0, 0)),
        scratch_shapes=[pltpu.VMEM((2, m, th * w), jnp.bfloat16)],
        compiler_params=pltpu.CompilerParams(
            dimension_semantics=("arbitrary",),
            vmem_limit_bytes=64 << 20,
        ),
    )(feats)


# speedup vs baseline: 1.0930x; 1.0930x over previous
import functools

import jax
import jax.numpy as jnp
from jax import lax
from jax.experimental import pallas as pl
from jax.experimental.pallas import tpu as pltpu

_TH = 32


def _gram_kernel(feat_ref, out_ref, *, nsteps, scale):
    kk = pl.program_id(0)

    @pl.when(kk == 0)
    def _():
        out_ref[...] = jnp.zeros_like(out_ref)

    f = feat_ref[...].astype(jnp.bfloat16)
    f = f.reshape(f.shape[0], f.shape[1] * f.shape[2])
    out_ref[...] += lax.dot_general(
        f, f,
        dimension_numbers=(((1,), (1,)), ((), ())),
        preferred_element_type=jnp.float32,
    )

    @pl.when(kk == nsteps - 1)
    def _():
        out_ref[...] = out_ref[...] * scale


def kernel(x):
    b, c, h, w = x.shape
    m = b * c
    feats = x.reshape(m, h, w)
    scale = 1.0 / float(b * c * h * w)

    th = _TH
    while th > 1 and h % th:
        th //= 2
    steps = h // th

    return pl.pallas_call(
        functools.partial(_gram_kernel, nsteps=steps, scale=scale),
        out_shape=jax.ShapeDtypeStruct((m, m), jnp.float32),
        grid=(steps,),
        in_specs=[pl.BlockSpec((m, th, w), lambda kk: (0, kk, 0))],
        out_specs=pl.BlockSpec((m, m), lambda kk: (0, 0)),
        compiler_params=pltpu.CompilerParams(
            dimension_semantics=("arbitrary",),
            vmem_limit_bytes=64 << 20,
        ),
    )(feats)
